# baseline (speedup 1.0000x reference)
import functools

import jax
import jax.numpy as jnp
from jax import lax
from jax.experimental import pallas as pl
from jax.experimental.pallas import tpu as pltpu
from jax.experimental.pallas import tpu_sc as plsc

N_POINTS = 1048576
NUM_WORKERS = 32
ALL_ROWS = N_POINTS // 128

SC_ROWS_PER_W = 32
SC_PTS_PER_W = SC_ROWS_PER_W * 128
SC_ROWS = SC_ROWS_PER_W * NUM_WORKERS
SC_PTS = SC_ROWS * 128
TC_ROWS = ALL_ROWS - SC_ROWS
TC_PTS = TC_ROWS * 128
TC_BLK_ROWS = 1024
TC_GRID = TC_ROWS // TC_BLK_ROWS

_CONST = 1.3862943611198906 - 1.8378770664093453 - 2.772588722239781

_LN2 = 0.6931471805599453
_SQRT2 = 1.4142135623730951
_LOG_POLY = (
    7.0376836292e-2, -1.1514610310e-1, 1.1676998740e-1, -1.2420140846e-1,
    1.4249322787e-1, -1.6668057665e-1, 2.0000714765e-1, -2.4999993993e-1,
    3.3333331174e-1,
)


def _sum_exp(v):
    u = v * jnp.float32(4.0)
    return (jnp.float32(1.0)
            + jnp.exp(u - jnp.float32(2.0))
            + jnp.exp(u + u - jnp.float32(8.0))
            + jnp.exp(u * jnp.float32(3.0) - jnp.float32(18.0)))


def _fast_log(s):
    bits = lax.bitcast_convert_type(s, jnp.int32)
    e = lax.shift_right_logical(bits, 23) - 127
    m = lax.bitcast_convert_type(
        jnp.bitwise_or(jnp.bitwise_and(bits, 0x007FFFFF), 0x3F800000),
        jnp.float32)
    big = m > jnp.float32(_SQRT2)
    m = jnp.where(big, m * jnp.float32(0.5), m)
    ef = e.astype(jnp.float32) + jnp.where(big, jnp.float32(1.0),
                                           jnp.float32(0.0))
    z = m - jnp.float32(1.0)
    r = jnp.float32(_LOG_POLY[0])
    for c in _LOG_POLY[1:]:
        r = r * z + jnp.float32(c)
    z2 = z * z
    y = z * z2 * r - jnp.float32(0.5) * z2 + z
    return y + ef * jnp.float32(_LN2)


def _axis_term_sc(v):
    return _fast_log(_sum_exp(v)) - jnp.float32(2.0) * v * v


_MESH = plsc.VectorSubcoreMesh(core_axis_name="c", subcore_axis_name="s")


@functools.partial(
    pl.kernel,
    mesh=_MESH,
    out_type=jax.ShapeDtypeStruct((SC_PTS,), jnp.float32),
    scratch_types=[
        pltpu.VMEM((SC_PTS_PER_W,), jnp.float32),
        pltpu.VMEM((SC_PTS_PER_W,), jnp.float32),
        pltpu.VMEM((SC_PTS_PER_W,), jnp.float32),
    ],
)
def _gmm_sc(xt_hbm, out_hbm, xv0, xv1, ov):
    wid = lax.axis_index("s") * 2 + lax.axis_index("c")
    p0 = TC_PTS + wid * SC_PTS_PER_W
    pltpu.sync_copy(xt_hbm.at[pl.ds(p0, SC_PTS_PER_W)], xv0)
    pltpu.sync_copy(xt_hbm.at[pl.ds(N_POINTS + p0, SC_PTS_PER_W)], xv1)

    def body(i, carry):
        for c in range(8):
            o = (i * 8 + c) * 16
            x0 = xv0[pl.ds(o, 16)]
            x1 = xv1[pl.ds(o, 16)]
            ov[pl.ds(o, 16)] = (_axis_term_sc(x0) + _axis_term_sc(x1)
                                + jnp.float32(_CONST))
        return carry

    lax.fori_loop(0, SC_PTS_PER_W // 128, body, 0)
    pltpu.sync_copy(ov, out_hbm.at[pl.ds(wid * SC_PTS_PER_W,
                                         SC_PTS_PER_W)])


def _tc_body(x0_ref, x1_ref, out):
    x0 = x0_ref[...]
    x1 = x1_ref[...]
    t = (jnp.log(_sum_exp(x0)) + jnp.log(_sum_exp(x1))
         - jnp.float32(2.0) * (x0 * x0 + x1 * x1))
    out[...] = t + jnp.float32(_CONST)


TC_BLK = TC_BLK_ROWS * 128


def _gmm_tc(xt_flat):
    return pl.pallas_call(
        _tc_body,
        grid=(TC_GRID,),
        in_specs=[
            pl.BlockSpec((TC_BLK,), lambda i: (i,)),
            pl.BlockSpec((TC_BLK,), lambda i: (N_POINTS // TC_BLK + i,)),
        ],
        out_specs=pl.BlockSpec((TC_BLK,), lambda i: (i,)),
        out_shape=jax.ShapeDtypeStruct((TC_PTS,), jnp.float32),
    )(xt_flat, xt_flat)


def kernel(x):
    xt = x.T.reshape(-1)
    tc_out = _gmm_tc(xt)
    sc_out = _gmm_sc(xt)
    return jnp.concatenate([tc_out, sc_out])


# speedup vs baseline: 1.0689x; 1.0689x over previous
import functools

import jax
import jax.numpy as jnp
from jax import lax
from jax.experimental import pallas as pl
from jax.experimental.pallas import tpu as pltpu
from jax.experimental.pallas import tpu_sc as plsc

N_POINTS = 1048576
NUM_WORKERS = 32
ALL_ROWS = N_POINTS // 128

SC_ROWS_PER_W = 32
SC_ROWS = SC_ROWS_PER_W * NUM_WORKERS
SC_PTS = SC_ROWS * 128
TC_ROWS = ALL_ROWS - SC_ROWS
TC_PTS = TC_ROWS * 128
TC_BLK_ROWS = 1024
TC_GRID = TC_ROWS // TC_BLK_ROWS

_CONST = 1.3862943611198906 - 1.8378770664093453 - 2.772588722239781

_LN2 = 0.6931471805599453
_SQRT2 = 1.4142135623730951
_LOG_POLY = (
    7.0376836292e-2, -1.1514610310e-1, 1.1676998740e-1, -1.2420140846e-1,
    1.4249322787e-1, -1.6668057665e-1, 2.0000714765e-1, -2.4999993993e-1,
    3.3333331174e-1,
)


def _sum_exp(v):
    u = v * jnp.float32(4.0)
    return (jnp.float32(1.0)
            + jnp.exp(u - jnp.float32(2.0))
            + jnp.exp(u + u - jnp.float32(8.0))
            + jnp.exp(u * jnp.float32(3.0) - jnp.float32(18.0)))


def _fast_log(s):
    bits = lax.bitcast_convert_type(s, jnp.int32)
    e = lax.shift_right_logical(bits, 23) - 127
    m = lax.bitcast_convert_type(
        jnp.bitwise_or(jnp.bitwise_and(bits, 0x007FFFFF), 0x3F800000),
        jnp.float32)
    big = m > jnp.float32(_SQRT2)
    m = jnp.where(big, m * jnp.float32(0.5), m)
    ef = e.astype(jnp.float32) + jnp.where(big, jnp.float32(1.0),
                                           jnp.float32(0.0))
    z = m - jnp.float32(1.0)
    r = jnp.float32(_LOG_POLY[0])
    for c in _LOG_POLY[1:]:
        r = r * z + jnp.float32(c)
    z2 = z * z
    y = z * z2 * r - jnp.float32(0.5) * z2 + z
    return y + ef * jnp.float32(_LN2)


def _axis_term_sc(v):
    return _fast_log(_sum_exp(v)) - jnp.float32(2.0) * v * v


_MESH = plsc.VectorSubcoreMesh(core_axis_name="c", subcore_axis_name="s")


@functools.partial(
    pl.kernel,
    mesh=_MESH,
    out_type=jax.ShapeDtypeStruct((SC_ROWS, 128), jnp.float32),
    scratch_types=[
        pltpu.VMEM((SC_ROWS_PER_W, 128), jnp.float32),
        pltpu.VMEM((SC_ROWS_PER_W, 128), jnp.float32),
        pltpu.VMEM((SC_ROWS_PER_W, 128), jnp.float32),
    ],
)
def _gmm_sc(xt_hbm, out_hbm, xv0, xv1, ov):
    wid = lax.axis_index("s") * 2 + lax.axis_index("c")
    r0 = TC_ROWS + wid * SC_ROWS_PER_W
    pltpu.sync_copy(xt_hbm.at[pl.ds(r0, SC_ROWS_PER_W)], xv0)
    pltpu.sync_copy(xt_hbm.at[pl.ds(ALL_ROWS + r0, SC_ROWS_PER_W)], xv1)

    def body(r, carry):
        for c in range(8):
            o = c * 16
            x0 = xv0[r, pl.ds(o, 16)]
            x1 = xv1[r, pl.ds(o, 16)]
            ov[r, pl.ds(o, 16)] = (_axis_term_sc(x0) + _axis_term_sc(x1)
                                   + jnp.float32(_CONST))
        return carry

    lax.fori_loop(0, SC_ROWS_PER_W, body, 0)
    pltpu.sync_copy(ov, out_hbm.at[pl.ds(wid * SC_ROWS_PER_W,
                                         SC_ROWS_PER_W)])


def _tc_body(x0_ref, x1_ref, out):
    x0 = x0_ref[...]
    x1 = x1_ref[...]
    t = (jnp.log(_sum_exp(x0)) + jnp.log(_sum_exp(x1))
         - jnp.float32(2.0) * (x0 * x0 + x1 * x1))
    out[...] = t + jnp.float32(_CONST)


def _gmm_tc(xt_rows):
    return pl.pallas_call(
        _tc_body,
        grid=(TC_GRID,),
        in_specs=[
            pl.BlockSpec((TC_BLK_ROWS, 128), lambda i: (i, 0)),
            pl.BlockSpec((TC_BLK_ROWS, 128),
                         lambda i: (ALL_ROWS // TC_BLK_ROWS + i, 0)),
        ],
        out_specs=pl.BlockSpec((TC_BLK_ROWS, 128), lambda i: (i, 0)),
        out_shape=jax.ShapeDtypeStruct((TC_ROWS, 128), jnp.float32),
    )(xt_rows, xt_rows)


def kernel(x):
    xt_rows = x.T.reshape(2 * ALL_ROWS, 128)
    tc_out = _gmm_tc(xt_rows)
    sc_out = _gmm_sc(xt_rows)
    return jnp.concatenate([tc_out, sc_out], axis=0).reshape(N_POINTS)
